# baseline (speedup 1.0000x reference)
import functools

import jax
import jax.numpy as jnp
from jax import lax
from jax.experimental import pallas as pl
from jax.experimental.pallas import tpu as pltpu
from jax.experimental.pallas import tpu_sc as plsc

VOCAB = 1_000_000
HIDDEN = 32
BATCH = 16384

NUM_CORES = 2
NUM_SUBCORES = 16
LANES = 16
NUM_WORKERS = NUM_CORES * NUM_SUBCORES
BPW = BATCH // NUM_WORKERS
GROUPS = BPW // LANES


def _sc_body(emb_hbm, ctx_hbm, iidx_hbm, cidx_hbm, out_hbm,
             iidx_v, cidx_v, erows_v, crows_v, out_v, sem_e, sem_c):
    wid = lax.axis_index("s") * NUM_CORES + lax.axis_index("c")
    base = wid * BPW

    pltpu.sync_copy(iidx_hbm.at[pl.ds(base, BPW)], iidx_v)
    pltpu.sync_copy(cidx_hbm.at[pl.ds(base, BPW)], cidx_v)

    cp_e = pltpu.async_copy(emb_hbm.at[iidx_v], erows_v, sem_e)
    cp_c = pltpu.async_copy(ctx_hbm.at[cidx_v], crows_v, sem_c)
    cp_e.wait()
    cp_c.wait()

    lanes = lax.iota(jnp.int32, LANES)

    def group(g, carry):
        rows = g * LANES + lanes
        acc = jnp.zeros((LANES,), jnp.float32)
        for d in range(HIDDEN):
            col = jnp.full((LANES,), d, jnp.int32)
            a = plsc.load_gather(erows_v, [rows, col])
            b = plsc.load_gather(crows_v, [rows, col])
            acc = acc + a * b
        out_v[pl.ds(g * LANES, LANES)] = 1.0 / (1.0 + jnp.exp(-acc))
        return carry

    lax.fori_loop(0, GROUPS, group, 0)

    pltpu.sync_copy(out_v, out_hbm.at[pl.ds(base, BPW)])


@jax.jit
def _sc_call(embedding, context, input_idxs, context_idxs):
    mesh = plsc.VectorSubcoreMesh(core_axis_name="c", subcore_axis_name="s")
    f = functools.partial(
        pl.kernel,
        out_type=jax.ShapeDtypeStruct((BATCH,), jnp.float32),
        mesh=mesh,
        compiler_params=pltpu.CompilerParams(
            needs_layout_passes=False, use_tc_tiling_on_sc=False),
        scratch_types=[
            pltpu.VMEM((BPW,), jnp.int32),
            pltpu.VMEM((BPW,), jnp.int32),
            pltpu.VMEM((BPW, HIDDEN), jnp.float32),
            pltpu.VMEM((BPW, HIDDEN), jnp.float32),
            pltpu.VMEM((BPW,), jnp.float32),
            pltpu.SemaphoreType.DMA,
            pltpu.SemaphoreType.DMA,
        ],
    )(_sc_body)
    return f(embedding, context, input_idxs, context_idxs)


def kernel(embedding, context, input_idxs, context_idxs):
    out = _sc_call(embedding, context,
                   input_idxs.astype(jnp.int32), context_idxs.astype(jnp.int32))
    return out.reshape(-1, 1)


# speedup vs baseline: 3.9751x; 3.9751x over previous
import functools

import jax
import jax.numpy as jnp
from jax import lax
from jax.experimental import pallas as pl
from jax.experimental.pallas import tpu as pltpu
from jax.experimental.pallas import tpu_sc as plsc

VOCAB = 1_000_000
HIDDEN = 32
BATCH = 16384

NUM_CORES = 2
NUM_SUBCORES = 16
LANES = 16
NUM_WORKERS = NUM_CORES * NUM_SUBCORES
BPW = BATCH // NUM_WORKERS
GSIZE = 4
NGROUPS = BPW // GSIZE
WIN = 128


def _sc_body(emb_hbm, ctx_hbm, iidx_hbm, cidx_hbm, out_hbm,
             iidx_v, cidx_v, dref, ebuf, cbuf, out_v,
             sem_e, sem_c):
    wid = lax.axis_index("s") * NUM_CORES + lax.axis_index("c")
    base = wid * BPW

    pltpu.sync_copy(iidx_hbm.at[pl.ds(base, BPW)], iidx_v.at[pl.ds(0, BPW)])
    pltpu.sync_copy(cidx_hbm.at[pl.ds(base, BPW)], cidx_v.at[pl.ds(0, BPW)])

    lanes = lax.iota(jnp.int32, LANES)
    dref[pl.ds(0, LANES)] = lanes
    dref[pl.ds(LANES, LANES)] = lanes + LANES

    def issue(g, slot_base):
        b0 = g * GSIZE
        vv_e = iidx_v[pl.ds(b0, LANES)]
        vv_c = cidx_v[pl.ds(b0, LANES)]
        for i in range(GSIZE):
            c0e = pl.multiple_of((vv_e[i] >> 7) << 7, WIN)
            pltpu.async_copy(
                emb_hbm.at[dref, pl.ds(c0e, WIN)], ebuf.at[slot_base + i], sem_e)
            c0c = pl.multiple_of((vv_c[i] >> 7) << 7, WIN)
            pltpu.async_copy(
                ctx_hbm.at[dref, pl.ds(c0c, WIN)], cbuf.at[slot_base + i], sem_c)

    def drain():
        for i in range(GSIZE):
            pltpu.make_async_copy(
                emb_hbm.at[dref, pl.ds(0, WIN)], ebuf.at[i], sem_e).wait()
            pltpu.make_async_copy(
                ctx_hbm.at[dref, pl.ds(0, WIN)], cbuf.at[i], sem_c).wait()

    def compute(g, slot_base):
        b0 = g * GSIZE
        cole = iidx_v[pl.ds(b0, LANES)] & (WIN - 1)
        colc = cidx_v[pl.ds(b0, LANES)] & (WIN - 1)
        slotv = slot_base + (lanes & (GSIZE - 1))
        acc = jnp.zeros((LANES,), jnp.float32)
        for d in range(HIDDEN):
            dv = jnp.full((LANES,), d, jnp.int32)
            a = plsc.load_gather(ebuf, [slotv, dv, cole])
            b = plsc.load_gather(cbuf, [slotv, dv, colc])
            acc = acc + a * b
        sig = 1.0 / (1.0 + jnp.exp(-acc))
        plsc.store_compressed(out_v.at[pl.ds(b0, LANES)], sig, mask=lanes < GSIZE)

    issue(0, 0)

    def step(g, carry):
        par = (g & 1) * GSIZE
        nxt = ((g + 1) & 1) * GSIZE
        issue(g + 1, nxt)
        drain()
        compute(g, par)
        return carry

    lax.fori_loop(0, NGROUPS - 1, step, 0)
    drain()
    compute(NGROUPS - 1, ((NGROUPS - 1) & 1) * GSIZE)

    pltpu.sync_copy(out_v.at[pl.ds(0, BPW)], out_hbm.at[pl.ds(base, BPW)])


@jax.jit
def _sc_call(emb_t, ctx_t, input_idxs, context_idxs):
    mesh = plsc.VectorSubcoreMesh(core_axis_name="c", subcore_axis_name="s")
    f = functools.partial(
        pl.kernel,
        out_type=jax.ShapeDtypeStruct((BATCH,), jnp.float32),
        mesh=mesh,
        compiler_params=pltpu.CompilerParams(needs_layout_passes=False),
        scratch_types=[
            pltpu.VMEM((BPW + LANES,), jnp.int32),
            pltpu.VMEM((BPW + LANES,), jnp.int32),
            pltpu.VMEM((HIDDEN,), jnp.int32),
            pltpu.VMEM((2 * GSIZE, HIDDEN, WIN), jnp.float32),
            pltpu.VMEM((2 * GSIZE, HIDDEN, WIN), jnp.float32),
            pltpu.VMEM((BPW + LANES,), jnp.float32),
            pltpu.SemaphoreType.DMA,
            pltpu.SemaphoreType.DMA,
        ],
    )(_sc_body)
    return f(emb_t, ctx_t, input_idxs, context_idxs)


def kernel(embedding, context, input_idxs, context_idxs):
    out = _sc_call(embedding.T, context.T,
                   input_idxs.astype(jnp.int32), context_idxs.astype(jnp.int32))
    return out.reshape(-1, 1)


# speedup vs baseline: 4.3414x; 1.0922x over previous
import functools

import jax
import jax.numpy as jnp
from jax import lax
from jax.experimental import pallas as pl
from jax.experimental.pallas import tpu as pltpu
from jax.experimental.pallas import tpu_sc as plsc

VOCAB = 1_000_000
HIDDEN = 32
BATCH = 16384

NUM_CORES = 2
NUM_SUBCORES = 16
LANES = 16
NUM_WORKERS = NUM_CORES * NUM_SUBCORES
BPW = BATCH // NUM_WORKERS
GSIZE = 4
NGROUPS = BPW // GSIZE
NBUF = 3
WIN = 128


def _sc_body(emb_hbm, ctx_hbm, iidx_hbm, cidx_hbm, out_hbm,
             iidx_v, cidx_v, dref, ebuf, cbuf, out_v,
             sem_e, sem_c):
    wid = lax.axis_index("s") * NUM_CORES + lax.axis_index("c")
    base = wid * BPW

    pltpu.sync_copy(iidx_hbm.at[pl.ds(base, BPW)], iidx_v.at[pl.ds(0, BPW)])
    pltpu.sync_copy(cidx_hbm.at[pl.ds(base, BPW)], cidx_v.at[pl.ds(0, BPW)])

    lanes = lax.iota(jnp.int32, LANES)
    dref[pl.ds(0, LANES)] = lanes
    dref[pl.ds(LANES, LANES)] = lanes + LANES

    def issue(g, slot_base):
        b0 = g * GSIZE
        vv_e = iidx_v[pl.ds(b0, LANES)]
        vv_c = cidx_v[pl.ds(b0, LANES)]
        for i in range(GSIZE):
            c0e = pl.multiple_of((vv_e[i] >> 7) << 7, WIN)
            pltpu.async_copy(
                emb_hbm.at[dref, pl.ds(c0e, WIN)], ebuf.at[slot_base + i], sem_e)
            c0c = pl.multiple_of((vv_c[i] >> 7) << 7, WIN)
            pltpu.async_copy(
                ctx_hbm.at[dref, pl.ds(c0c, WIN)], cbuf.at[slot_base + i], sem_c)

    def drain():
        for i in range(GSIZE):
            pltpu.make_async_copy(
                emb_hbm.at[dref, pl.ds(0, WIN)], ebuf.at[i], sem_e).wait()
            pltpu.make_async_copy(
                ctx_hbm.at[dref, pl.ds(0, WIN)], cbuf.at[i], sem_c).wait()

    def compute(g, slot_base):
        b0 = g * GSIZE
        cole = iidx_v[pl.ds(b0, LANES)] & (WIN - 1)
        colc = cidx_v[pl.ds(b0, LANES)] & (WIN - 1)
        slotv = slot_base + (lanes & (GSIZE - 1))
        acc = jnp.zeros((LANES,), jnp.float32)
        for d in range(HIDDEN):
            dv = jnp.full((LANES,), d, jnp.int32)
            a = plsc.load_gather(ebuf, [slotv, dv, cole])
            b = plsc.load_gather(cbuf, [slotv, dv, colc])
            acc = acc + a * b
        sig = 1.0 / (1.0 + jnp.exp(-acc))
        plsc.store_compressed(out_v.at[pl.ds(b0, LANES)], sig, mask=lanes < GSIZE)

    def slot(g):
        return lax.rem(g, NBUF) * GSIZE

    issue(0, slot(0))
    issue(1, slot(1))

    def step(g, carry):
        issue(g + 2, slot(g + 2))
        drain()
        compute(g, slot(g))
        return carry

    lax.fori_loop(0, NGROUPS - 2, step, 0)
    drain()
    compute(NGROUPS - 2, slot(NGROUPS - 2))
    drain()
    compute(NGROUPS - 1, slot(NGROUPS - 1))

    pltpu.sync_copy(out_v.at[pl.ds(0, BPW)], out_hbm.at[pl.ds(base, BPW)])


@jax.jit
def _sc_call(emb_t, ctx_t, input_idxs, context_idxs):
    mesh = plsc.VectorSubcoreMesh(core_axis_name="c", subcore_axis_name="s")
    f = functools.partial(
        pl.kernel,
        out_type=jax.ShapeDtypeStruct((BATCH,), jnp.float32),
        mesh=mesh,
        compiler_params=pltpu.CompilerParams(needs_layout_passes=False),
        scratch_types=[
            pltpu.VMEM((BPW + LANES,), jnp.int32),
            pltpu.VMEM((BPW + LANES,), jnp.int32),
            pltpu.VMEM((HIDDEN,), jnp.int32),
            pltpu.VMEM((NBUF * GSIZE, HIDDEN, WIN), jnp.float32),
            pltpu.VMEM((NBUF * GSIZE, HIDDEN, WIN), jnp.float32),
            pltpu.VMEM((BPW + LANES,), jnp.float32),
            pltpu.SemaphoreType.DMA,
            pltpu.SemaphoreType.DMA,
        ],
    )(_sc_body)
    return f(emb_t, ctx_t, input_idxs, context_idxs)


def kernel(embedding, context, input_idxs, context_idxs):
    out = _sc_call(embedding.T, context.T,
                   input_idxs.astype(jnp.int32), context_idxs.astype(jnp.int32))
    return out.reshape(-1, 1)


# speedup vs baseline: 4.4427x; 1.0233x over previous
import functools

import jax
import jax.numpy as jnp
from jax import lax
from jax.experimental import pallas as pl
from jax.experimental.pallas import tpu as pltpu
from jax.experimental.pallas import tpu_sc as plsc

VOCAB = 1_000_000
HIDDEN = 32
BATCH = 16384

NUM_CORES = 2
NUM_SUBCORES = 16
LANES = 16
NUM_WORKERS = NUM_CORES * NUM_SUBCORES
BPW = BATCH // NUM_WORKERS
GSIZE = 4
NGROUPS = BPW // GSIZE
NBUF = 3
WIN = 128


DMAJ = 4


def _sc_body(emb_hbm, ctx_hbm, iidx_hbm, cidx_hbm, out_hbm,
             iidx_v, cidx_v, dref, ebuf, cbuf, out_v,
             sem_e, sem_c):
    wid = lax.axis_index("s") * NUM_CORES + lax.axis_index("c")
    base = wid * BPW

    pltpu.sync_copy(iidx_hbm.at[pl.ds(base, BPW)], iidx_v.at[pl.ds(0, BPW)])
    pltpu.sync_copy(cidx_hbm.at[pl.ds(base, BPW)], cidx_v.at[pl.ds(0, BPW)])

    lanes = lax.iota(jnp.int32, LANES)
    dref[pl.ds(0, LANES)] = lanes

    def issue(g, slot_base):
        b0 = g * GSIZE
        vv_e = iidx_v[pl.ds(b0, LANES)]
        vv_c = cidx_v[pl.ds(b0, LANES)]
        for i in range(GSIZE):
            c0e = pl.multiple_of((vv_e[i] >> 7) << 7, WIN)
            pltpu.async_copy(
                emb_hbm.at[dref.at[pl.ds(0, DMAJ)], :, pl.ds(c0e, WIN)],
                ebuf.at[slot_base + i], sem_e)
            c0c = pl.multiple_of((vv_c[i] >> 7) << 7, WIN)
            pltpu.async_copy(
                ctx_hbm.at[dref.at[pl.ds(0, DMAJ)], :, pl.ds(c0c, WIN)],
                cbuf.at[slot_base + i], sem_c)

    def drain():
        for i in range(GSIZE):
            pltpu.make_async_copy(
                emb_hbm.at[dref.at[pl.ds(0, DMAJ)], :, pl.ds(0, WIN)],
                ebuf.at[i], sem_e).wait()
            pltpu.make_async_copy(
                ctx_hbm.at[dref.at[pl.ds(0, DMAJ)], :, pl.ds(0, WIN)],
                cbuf.at[i], sem_c).wait()

    def compute(g, slot_base):
        b0 = g * GSIZE
        cole = iidx_v[pl.ds(b0, LANES)] & (WIN - 1)
        colc = cidx_v[pl.ds(b0, LANES)] & (WIN - 1)
        slotv = slot_base + (lanes & (GSIZE - 1))
        acc = jnp.zeros((LANES,), jnp.float32)
        for d in range(HIDDEN):
            dhi = jnp.full((LANES,), d >> 3, jnp.int32)
            dlo = jnp.full((LANES,), d & 7, jnp.int32)
            a = plsc.load_gather(ebuf, [slotv, dhi, dlo, cole])
            b = plsc.load_gather(cbuf, [slotv, dhi, dlo, colc])
            acc = acc + a * b
        sig = 1.0 / (1.0 + jnp.exp(-acc))
        plsc.store_compressed(out_v.at[pl.ds(b0, LANES)], sig, mask=lanes < GSIZE)

    def slot(g):
        return lax.rem(g, NBUF) * GSIZE

    issue(0, slot(0))
    issue(1, slot(1))

    def step(g, carry):
        issue(g + 2, slot(g + 2))
        drain()
        compute(g, slot(g))
        return carry

    lax.fori_loop(0, NGROUPS - 2, step, 0)
    drain()
    compute(NGROUPS - 2, slot(NGROUPS - 2))
    drain()
    compute(NGROUPS - 1, slot(NGROUPS - 1))

    pltpu.sync_copy(out_v.at[pl.ds(0, BPW)], out_hbm.at[pl.ds(base, BPW)])


@jax.jit
def _sc_call(emb_t, ctx_t, input_idxs, context_idxs):
    mesh = plsc.VectorSubcoreMesh(core_axis_name="c", subcore_axis_name="s")
    f = functools.partial(
        pl.kernel,
        out_type=jax.ShapeDtypeStruct((BATCH,), jnp.float32),
        mesh=mesh,
        compiler_params=pltpu.CompilerParams(needs_layout_passes=False),
        scratch_types=[
            pltpu.VMEM((BPW + LANES,), jnp.int32),
            pltpu.VMEM((BPW + LANES,), jnp.int32),
            pltpu.VMEM((LANES,), jnp.int32),
            pltpu.VMEM((NBUF * GSIZE, DMAJ, 8, WIN), jnp.float32),
            pltpu.VMEM((NBUF * GSIZE, DMAJ, 8, WIN), jnp.float32),
            pltpu.VMEM((BPW + LANES,), jnp.float32),
            pltpu.SemaphoreType.DMA,
            pltpu.SemaphoreType.DMA,
        ],
    )(_sc_body)
    return f(emb_t, ctx_t, input_idxs, context_idxs)


def kernel(embedding, context, input_idxs, context_idxs):
    out = _sc_call(embedding.T.reshape(4, 8, VOCAB), context.T.reshape(4, 8, VOCAB),
                   input_idxs.astype(jnp.int32), context_idxs.astype(jnp.int32))
    return out.reshape(-1, 1)


# speedup vs baseline: 4.4769x; 1.0077x over previous
import functools

import jax
import jax.numpy as jnp
from jax import lax
from jax.experimental import pallas as pl
from jax.experimental.pallas import tpu as pltpu
from jax.experimental.pallas import tpu_sc as plsc

VOCAB = 1_000_000
HIDDEN = 32
BATCH = 16384

NUM_CORES = 2
NUM_SUBCORES = 16
LANES = 16
NUM_WORKERS = NUM_CORES * NUM_SUBCORES
BPW = BATCH // NUM_WORKERS
GSIZE = 4
NGROUPS = BPW // GSIZE
NBUF = 3
WIN = 128


DMAJ = 4


def _sc_body(emb_hbm, ctx_hbm, iidx_hbm, cidx_hbm, out_hbm,
             iidx_v, cidx_v, ebuf, cbuf, out_v,
             sem_e, sem_c):
    wid = lax.axis_index("s") * NUM_CORES + lax.axis_index("c")
    base = wid * BPW

    pltpu.sync_copy(iidx_hbm.at[pl.ds(base, BPW)], iidx_v.at[pl.ds(0, BPW)])
    pltpu.sync_copy(cidx_hbm.at[pl.ds(base, BPW)], cidx_v.at[pl.ds(0, BPW)])

    lanes = lax.iota(jnp.int32, LANES)

    def issue(g, slot_base):
        b0 = g * GSIZE
        vv_e = iidx_v[pl.ds(b0, LANES)]
        vv_c = cidx_v[pl.ds(b0, LANES)]
        for i in range(GSIZE):
            c0e = pl.multiple_of((vv_e[i] >> 7) << 7, WIN)
            pltpu.async_copy(
                emb_hbm.at[:, :, pl.ds(c0e, WIN)],
                ebuf.at[slot_base + i], sem_e)
            c0c = pl.multiple_of((vv_c[i] >> 7) << 7, WIN)
            pltpu.async_copy(
                ctx_hbm.at[:, :, pl.ds(c0c, WIN)],
                cbuf.at[slot_base + i], sem_c)

    def drain():
        for i in range(GSIZE):
            pltpu.make_async_copy(
                emb_hbm.at[:, :, pl.ds(0, WIN)], ebuf.at[i], sem_e).wait()
            pltpu.make_async_copy(
                ctx_hbm.at[:, :, pl.ds(0, WIN)], cbuf.at[i], sem_c).wait()

    def compute(g, slot_base):
        b0 = g * GSIZE
        cole = iidx_v[pl.ds(b0, LANES)] & (WIN - 1)
        colc = cidx_v[pl.ds(b0, LANES)] & (WIN - 1)
        slotv = slot_base + (lanes & (GSIZE - 1))
        acc = jnp.zeros((LANES,), jnp.float32)
        for d in range(HIDDEN):
            dhi = jnp.full((LANES,), d >> 3, jnp.int32)
            dlo = jnp.full((LANES,), d & 7, jnp.int32)
            a = plsc.load_gather(ebuf, [slotv, dhi, dlo, cole])
            b = plsc.load_gather(cbuf, [slotv, dhi, dlo, colc])
            acc = acc + a * b
        sig = 1.0 / (1.0 + jnp.exp(-acc))
        plsc.store_compressed(out_v.at[pl.ds(b0, LANES)], sig, mask=lanes < GSIZE)

    def slot(g):
        return lax.rem(g, NBUF) * GSIZE

    issue(0, slot(0))
    issue(1, slot(1))

    def step(g, carry):
        issue(g + 2, slot(g + 2))
        drain()
        compute(g, slot(g))
        return carry

    lax.fori_loop(0, NGROUPS - 2, step, 0)
    drain()
    compute(NGROUPS - 2, slot(NGROUPS - 2))
    drain()
    compute(NGROUPS - 1, slot(NGROUPS - 1))

    pltpu.sync_copy(out_v.at[pl.ds(0, BPW)], out_hbm.at[pl.ds(base, BPW)])


@jax.jit
def _sc_call(emb_t, ctx_t, input_idxs, context_idxs):
    mesh = plsc.VectorSubcoreMesh(core_axis_name="c", subcore_axis_name="s")
    f = functools.partial(
        pl.kernel,
        out_type=jax.ShapeDtypeStruct((BATCH,), jnp.float32),
        mesh=mesh,
        compiler_params=pltpu.CompilerParams(needs_layout_passes=False),
        scratch_types=[
            pltpu.VMEM((BPW + LANES,), jnp.int32),
            pltpu.VMEM((BPW + LANES,), jnp.int32),
            pltpu.VMEM((NBUF * GSIZE, DMAJ, 8, WIN), jnp.float32),
            pltpu.VMEM((NBUF * GSIZE, DMAJ, 8, WIN), jnp.float32),
            pltpu.VMEM((BPW + LANES,), jnp.float32),
            pltpu.SemaphoreType.DMA,
            pltpu.SemaphoreType.DMA,
        ],
    )(_sc_body)
    return f(emb_t, ctx_t, input_idxs, context_idxs)


def kernel(embedding, context, input_idxs, context_idxs):
    out = _sc_call(embedding.T.reshape(4, 8, VOCAB), context.T.reshape(4, 8, VOCAB),
                   input_idxs.astype(jnp.int32), context_idxs.astype(jnp.int32))
    return out.reshape(-1, 1)
